# Initial kernel scaffold; baseline (speedup 1.0000x reference)
import jax
import jax.numpy as jnp
from jax.experimental import pallas as pl


def kernel(x, edge_index, edge_weight, W):
    raise NotImplementedError("write your pallas kernel here")


# speedup vs baseline: 4.4184x; 4.4184x over previous
import functools

import jax
import jax.numpy as jnp
from jax import lax
from jax.experimental import pallas as pl
from jax.experimental.pallas import tpu as pltpu
from jax.experimental.pallas import tpu_sc as plsc

N_NODES = 10000
N_EDGES = 320000
D_IN = 128
D_OUT = 128

NUM_CORES = 2
NUM_SUBCORES = 16
EDGES_PER_CORE = N_EDGES // NUM_CORES
EDGES_PER_TILE = EDGES_PER_CORE // NUM_SUBCORES
CHUNK = 80
N_CHUNKS = EDGES_PER_TILE // CHUNK

ROWS_PER_TILE = 624
TAIL_ROW0 = NUM_SUBCORES * ROWS_PER_TILE
TAIL_ROWS = N_NODES - TAIL_ROW0
STAGE_ROWS = 104
N_STAGE = ROWS_PER_TILE // STAGE_ROWS


def _matmul_body(x_ref, w_ref, o_ref):
    o_ref[...] = jax.lax.dot_general(
        x_ref[...], w_ref[...], (((1,), (0,)), ((), ())),
        precision=jax.lax.Precision.HIGHEST,
        preferred_element_type=jnp.float32,
    )


def _support(x, W):
    return pl.pallas_call(
        _matmul_body,
        grid=(10,),
        in_specs=[
            pl.BlockSpec((N_NODES // 10, D_IN), lambda i: (i, 0)),
            pl.BlockSpec((D_IN, D_OUT), lambda i: (0, 0)),
        ],
        out_specs=pl.BlockSpec((N_NODES // 10, D_OUT), lambda i: (i, 0)),
        out_shape=jax.ShapeDtypeStruct((N_NODES, D_OUT), jnp.float32),
    )(x, W)


def _sc_spmm(sup, src, dst, ew):
    mesh = plsc.VectorSubcoreMesh(core_axis_name="c", subcore_axis_name="s")

    @functools.partial(
        pl.kernel,
        mesh=mesh,
        out_type=jax.ShapeDtypeStruct((NUM_CORES, N_NODES, D_OUT), jnp.float32),
        scratch_types=[
            pltpu.VMEM((CHUNK,), jnp.int32),
            pltpu.VMEM((CHUNK,), jnp.int32),
            pltpu.VMEM((CHUNK,), jnp.float32),
            pltpu.VMEM((CHUNK, D_OUT), jnp.float32),
            pltpu.VMEM((STAGE_ROWS, D_OUT), jnp.float32),
            pltpu.VMEM_SHARED((N_NODES, D_OUT), jnp.float32),
            pltpu.SemaphoreType.DMA,
        ],
    )
    def k(sup_hbm, src_hbm, dst_hbm, ew_hbm, out_hbm,
          src_v, dst_v, ew_v, rows_v, stage_v, acc, sem):
        cid = lax.axis_index("c")
        sid = lax.axis_index("s")

        zeros16 = jnp.zeros((16,), jnp.float32)

        def zero_body(r, carry):
            for j in range(D_OUT // 16):
                stage_v[r, pl.ds(j * 16, 16)] = zeros16
            return carry

        lax.fori_loop(0, STAGE_ROWS, zero_body, 0)
        row0 = sid * ROWS_PER_TILE
        for kk in range(N_STAGE):
            pltpu.sync_copy(
                stage_v, acc.at[pl.ds(row0 + kk * STAGE_ROWS, STAGE_ROWS)])

        @pl.when(sid == NUM_SUBCORES - 1)
        def _zero_tail():
            pltpu.sync_copy(stage_v.at[pl.ds(0, TAIL_ROWS)],
                            acc.at[pl.ds(TAIL_ROW0, TAIL_ROWS)])

        plsc.subcore_barrier()

        def chunk_body(g, carry):
            base = cid * EDGES_PER_CORE + sid * EDGES_PER_TILE + g * CHUNK
            pltpu.sync_copy(src_hbm.at[pl.ds(base, CHUNK)], src_v)
            pltpu.sync_copy(dst_hbm.at[pl.ds(base, CHUNK)], dst_v)
            pltpu.sync_copy(ew_hbm.at[pl.ds(base, CHUNK)], ew_v)
            pltpu.async_copy(sup_hbm.at[src_v], rows_v, sem).wait()

            def mul_group(g16, c2):
                gb = g16 * 16
                w16 = ew_v[pl.ds(gb, 16)]
                for r in range(16):
                    e = gb + r
                    wv = w16[r]
                    for j in range(D_OUT // 16):
                        sl = pl.ds(j * 16, 16)
                        rows_v[e, sl] = rows_v[e, sl] * wv
                return c2

            lax.fori_loop(0, CHUNK // 16, mul_group, 0)
            pltpu.sync_copy(rows_v, acc.at[dst_v], add=True)
            return carry

        lax.fori_loop(0, N_CHUNKS, chunk_body, 0)
        plsc.subcore_barrier()

        for kk in range(N_STAGE):
            r0 = row0 + kk * STAGE_ROWS
            pltpu.sync_copy(acc.at[pl.ds(r0, STAGE_ROWS)], stage_v)
            pltpu.sync_copy(stage_v, out_hbm.at[cid].at[pl.ds(r0, STAGE_ROWS)])

        @pl.when(sid == NUM_SUBCORES - 1)
        def _write_tail():
            pltpu.sync_copy(acc.at[pl.ds(TAIL_ROW0, TAIL_ROWS)],
                            stage_v.at[pl.ds(0, TAIL_ROWS)])
            pltpu.sync_copy(
                stage_v.at[pl.ds(0, TAIL_ROWS)],
                out_hbm.at[cid].at[pl.ds(TAIL_ROW0, TAIL_ROWS)])

    return k(sup, src, dst, ew)


def _combine_body(p_ref, o_ref):
    o_ref[...] = p_ref[0] + p_ref[1]


def _combine(partials):
    return pl.pallas_call(
        _combine_body,
        grid=(10,),
        in_specs=[pl.BlockSpec((NUM_CORES, N_NODES // 10, D_OUT),
                               lambda i: (0, i, 0))],
        out_specs=pl.BlockSpec((N_NODES // 10, D_OUT), lambda i: (i, 0)),
        out_shape=jax.ShapeDtypeStruct((N_NODES, D_OUT), jnp.float32),
    )(partials)


def kernel(x, edge_index, edge_weight, W):
    src = edge_index[0].astype(jnp.int32)
    dst = edge_index[1].astype(jnp.int32)
    sup = _support(x, W)
    return _combine(_sc_spmm(sup, src, dst, edge_weight))


# speedup vs baseline: 8.8021x; 1.9921x over previous
import functools

import jax
import jax.numpy as jnp
from jax import lax
from jax.experimental import pallas as pl
from jax.experimental.pallas import tpu as pltpu
from jax.experimental.pallas import tpu_sc as plsc

N_NODES = 10000
N_EDGES = 320000
D_IN = 128
D_OUT = 128

NUM_CORES = 2
NUM_SUBCORES = 16
EDGES_PER_CORE = N_EDGES // NUM_CORES
CHUNK = 128
CHUNKS_PER_TILE = 78
TILE_EDGES = CHUNK * CHUNKS_PER_TILE
LEFTOVER0 = NUM_SUBCORES * TILE_EDGES

ROWS_PER_TILE = 624
TAIL_ROW0 = NUM_SUBCORES * ROWS_PER_TILE
TAIL_ROWS = N_NODES - TAIL_ROW0
STAGE_ROWS = 104
N_STAGE = ROWS_PER_TILE // STAGE_ROWS


def _matmul_body(x_ref, w_ref, o_ref):
    o_ref[...] = jax.lax.dot_general(
        x_ref[...], w_ref[...], (((1,), (0,)), ((), ())),
        precision=jax.lax.Precision.HIGHEST,
        preferred_element_type=jnp.float32,
    )


def _support(x, W):
    return pl.pallas_call(
        _matmul_body,
        grid=(10,),
        in_specs=[
            pl.BlockSpec((N_NODES // 10, D_IN), lambda i: (i, 0)),
            pl.BlockSpec((D_IN, D_OUT), lambda i: (0, 0)),
        ],
        out_specs=pl.BlockSpec((N_NODES // 10, D_OUT), lambda i: (i, 0)),
        out_shape=jax.ShapeDtypeStruct((N_NODES, D_OUT), jnp.float32),
    )(x, W)


def _sc_spmm(sup, src, dst, ew):
    mesh = plsc.VectorSubcoreMesh(core_axis_name="c", subcore_axis_name="s")

    @functools.partial(
        pl.kernel,
        mesh=mesh,
        out_type=jax.ShapeDtypeStruct((NUM_CORES, N_NODES, D_OUT), jnp.float32),
        scratch_types=[
            pltpu.VMEM((CHUNK,), jnp.int32),
            pltpu.VMEM((CHUNK,), jnp.int32),
            pltpu.VMEM((CHUNK,), jnp.float32),
            pltpu.VMEM((CHUNK,), jnp.int32),
            pltpu.VMEM((CHUNK,), jnp.int32),
            pltpu.VMEM((CHUNK,), jnp.float32),
            pltpu.VMEM((CHUNK, D_OUT), jnp.float32),
            pltpu.VMEM((CHUNK, D_OUT), jnp.float32),
            pltpu.VMEM_SHARED((N_NODES, D_OUT), jnp.float32),
            pltpu.SemaphoreType.DMA,
            pltpu.SemaphoreType.DMA,
            pltpu.SemaphoreType.DMA,
            pltpu.SemaphoreType.DMA,
        ],
    )
    def k(sup_hbm, src_hbm, dst_hbm, ew_hbm, out_hbm,
          src_a, dst_a, ew_a, src_b, dst_b, ew_b, rows_a, rows_b, acc,
          ga, gb, ia, ib):
        cid = lax.axis_index("c")
        sid = lax.axis_index("s")

        corebase = cid * EDGES_PER_CORE
        tilebase = corebase + sid * TILE_EDGES
        npairs = jnp.where(sid == 0, CHUNKS_PER_TILE // 2 + 1,
                           CHUNKS_PER_TILE // 2)
        cmax = 2 * npairs - 1

        def chunk_base(c):
            return jnp.where(
                c < CHUNKS_PER_TILE,
                tilebase + c * CHUNK,
                corebase + LEFTOVER0 + (c - CHUNKS_PER_TILE) * CHUNK)

        def load_idx(c, s_v, d_v, w_v, sem):
            base = chunk_base(c)
            cps = [
                pltpu.async_copy(src_hbm.at[pl.ds(base, CHUNK)], s_v, sem),
                pltpu.async_copy(dst_hbm.at[pl.ds(base, CHUNK)], d_v, sem),
                pltpu.async_copy(ew_hbm.at[pl.ds(base, CHUNK)], w_v, sem),
            ]
            return cps

        def wait_idx(cps):
            for cp in cps:
                cp.wait()

        def mul_rows(rows_v, ew_v):
            def mul_group(g16, c2):
                gbase = g16 * 16
                w16 = ew_v[pl.ds(gbase, 16)]
                for r in range(16):
                    e = gbase + r
                    wv = w16[r]
                    for j in range(D_OUT // 16):
                        sl = pl.ds(j * 16, 16)
                        rows_v[e, sl] = rows_v[e, sl] * wv
                return c2

            lax.fori_loop(0, CHUNK // 16, mul_group, 0)

        wait_idx(load_idx(0, src_a, dst_a, ew_a, ia))
        cp_ga = [pltpu.async_copy(sup_hbm.at[src_a], rows_a, ga)]
        cps_ib = load_idx(1, src_b, dst_b, ew_b, ib)

        zeros16 = jnp.zeros((16,), jnp.float32)

        def zero_body(r, carry):
            for j in range(D_OUT // 16):
                rows_b[r, pl.ds(j * 16, 16)] = zeros16
            return carry

        lax.fori_loop(0, CHUNK, zero_body, 0)
        row0 = sid * ROWS_PER_TILE
        for kk in range(ROWS_PER_TILE // CHUNK + 1):
            nrows = CHUNK if kk < ROWS_PER_TILE // CHUNK else ROWS_PER_TILE % CHUNK
            pltpu.sync_copy(rows_b.at[pl.ds(0, nrows)],
                            acc.at[pl.ds(row0 + kk * CHUNK, nrows)])

        @pl.when(sid == NUM_SUBCORES - 1)
        def _zero_tail():
            pltpu.sync_copy(rows_b.at[pl.ds(0, TAIL_ROWS)],
                            acc.at[pl.ds(TAIL_ROW0, TAIL_ROWS)])

        plsc.subcore_barrier()

        def pair_body(j, carry):
            c_a = 2 * j
            c_b = c_a + 1
            wait_idx(cps_ib)
            cp_gb = pltpu.async_copy(sup_hbm.at[src_b], rows_b, gb)
            wait_idx(cp_ga)
            mul_rows(rows_a, ew_a)
            pltpu.sync_copy(rows_a, acc.at[dst_a], add=True)
            cps_ia = load_idx(jnp.minimum(c_a + 2, cmax), src_a, dst_a,
                              ew_a, ia)
            cp_gb.wait()
            mul_rows(rows_b, ew_b)
            pltpu.sync_copy(rows_b, acc.at[dst_b], add=True)
            cps_ib2 = load_idx(jnp.minimum(c_b + 2, cmax), src_b, dst_b,
                               ew_b, ib)
            wait_idx(cps_ia)
            cp_ga2 = pltpu.async_copy(
                sup_hbm.at[src_a], rows_a, ga)
            return carry

        lax.fori_loop(0, npairs, pair_body, 0)
        pltpu.make_async_copy(sup_hbm.at[src_a], rows_a, ga).wait()
        pltpu.make_async_copy(src_hbm.at[pl.ds(0, CHUNK)], src_b, ib).wait()
        pltpu.make_async_copy(dst_hbm.at[pl.ds(0, CHUNK)], dst_b, ib).wait()
        pltpu.make_async_copy(ew_hbm.at[pl.ds(0, CHUNK)], ew_b, ib).wait()
        plsc.subcore_barrier()

        for kk in range(ROWS_PER_TILE // CHUNK + 1):
            nrows = CHUNK if kk < ROWS_PER_TILE // CHUNK else ROWS_PER_TILE % CHUNK
            r0 = row0 + kk * CHUNK
            pltpu.sync_copy(acc.at[pl.ds(r0, nrows)],
                            rows_a.at[pl.ds(0, nrows)])
            pltpu.sync_copy(rows_a.at[pl.ds(0, nrows)],
                            out_hbm.at[cid].at[pl.ds(r0, nrows)])

        @pl.when(sid == NUM_SUBCORES - 1)
        def _write_tail():
            pltpu.sync_copy(acc.at[pl.ds(TAIL_ROW0, TAIL_ROWS)],
                            rows_a.at[pl.ds(0, TAIL_ROWS)])
            pltpu.sync_copy(
                rows_a.at[pl.ds(0, TAIL_ROWS)],
                out_hbm.at[cid].at[pl.ds(TAIL_ROW0, TAIL_ROWS)])

    return k(sup, src, dst, ew)


def _combine_body(p_ref, o_ref):
    o_ref[...] = p_ref[0] + p_ref[1]


def _combine(partials):
    return pl.pallas_call(
        _combine_body,
        grid=(10,),
        in_specs=[pl.BlockSpec((NUM_CORES, N_NODES // 10, D_OUT),
                               lambda i: (0, i, 0))],
        out_specs=pl.BlockSpec((N_NODES // 10, D_OUT), lambda i: (i, 0)),
        out_shape=jax.ShapeDtypeStruct((N_NODES, D_OUT), jnp.float32),
    )(partials)


def kernel(x, edge_index, edge_weight, W):
    src = edge_index[0].astype(jnp.int32)
    dst = edge_index[1].astype(jnp.int32)
    sup = _support(x, W)
    return _combine(_sc_spmm(sup, src, dst, edge_weight))


# speedup vs baseline: 10.4186x; 1.1837x over previous
import functools

import jax
import jax.numpy as jnp
from jax import lax
from jax.experimental import pallas as pl
from jax.experimental.pallas import tpu as pltpu
from jax.experimental.pallas import tpu_sc as plsc

N_NODES = 10000
N_EDGES = 320000
D_IN = 128
D_OUT = 128

NUM_CORES = 2
NUM_SUBCORES = 16
EDGES_PER_CORE = N_EDGES // NUM_CORES
CHUNK = 128
CHUNKS_PER_TILE = 78
TILE_EDGES = CHUNK * CHUNKS_PER_TILE
LEFTOVER0 = NUM_SUBCORES * TILE_EDGES

ROWS_PER_TILE = 624
TAIL_ROW0 = NUM_SUBCORES * ROWS_PER_TILE
TAIL_ROWS = N_NODES - TAIL_ROW0
STAGE_ROWS = 104
N_STAGE = ROWS_PER_TILE // STAGE_ROWS


def _matmul_body(x_ref, w_ref, o_ref):
    o_ref[...] = jax.lax.dot_general(
        x_ref[...], w_ref[...], (((1,), (0,)), ((), ())),
        precision=jax.lax.Precision.HIGHEST,
        preferred_element_type=jnp.float32,
    )


def _support(x, W):
    return pl.pallas_call(
        _matmul_body,
        grid=(10,),
        in_specs=[
            pl.BlockSpec((N_NODES // 10, D_IN), lambda i: (i, 0)),
            pl.BlockSpec((D_IN, D_OUT), lambda i: (0, 0)),
        ],
        out_specs=pl.BlockSpec((N_NODES // 10, D_OUT), lambda i: (i, 0)),
        out_shape=jax.ShapeDtypeStruct((N_NODES, D_OUT), jnp.float32),
    )(x, W)


def _sc_spmm(sup, src, dst, ew):
    mesh = plsc.VectorSubcoreMesh(core_axis_name="c", subcore_axis_name="s")

    @functools.partial(
        pl.kernel,
        mesh=mesh,
        out_type=jax.ShapeDtypeStruct((NUM_CORES, N_NODES, D_OUT), jnp.float32),
        scratch_types=[
            pltpu.VMEM((CHUNK,), jnp.int32),
            pltpu.VMEM((CHUNK,), jnp.int32),
            pltpu.VMEM((CHUNK,), jnp.float32),
            pltpu.VMEM((CHUNK,), jnp.int32),
            pltpu.VMEM((CHUNK,), jnp.int32),
            pltpu.VMEM((CHUNK,), jnp.float32),
            pltpu.VMEM((CHUNK, D_OUT), jnp.float32),
            pltpu.VMEM((CHUNK, D_OUT), jnp.float32),
            pltpu.VMEM((2, CHUNK // 2), jnp.int32),
            pltpu.VMEM((2, CHUNK // 2), jnp.int32),
            pltpu.VMEM_SHARED((N_NODES, D_OUT), jnp.float32),
            pltpu.SemaphoreType.DMA,
            pltpu.SemaphoreType.DMA,
            pltpu.SemaphoreType.DMA,
            pltpu.SemaphoreType.DMA,
            pltpu.SemaphoreType.DMA,
            pltpu.SemaphoreType.DMA,
        ],
    )
    def k(sup_hbm, src_hbm, dst_hbm, ew_hbm, out_hbm,
          src_a, dst_a, ew_a, src_b, dst_b, ew_b, rows_a, rows_b,
          dsc_a, dsc_b, acc, ga, gb, ia, ib, sa, sb):
        cid = lax.axis_index("c")
        sid = lax.axis_index("s")

        corebase = cid * EDGES_PER_CORE
        tilebase = corebase + sid * TILE_EDGES
        npairs = jnp.where(sid == 0, CHUNKS_PER_TILE // 2 + 1,
                           CHUNKS_PER_TILE // 2)
        cmax = 2 * npairs - 1

        def chunk_base(c):
            return jnp.where(
                c < CHUNKS_PER_TILE,
                tilebase + c * CHUNK,
                corebase + LEFTOVER0 + (c - CHUNKS_PER_TILE) * CHUNK)

        def load_idx(c, s_v, d_v, w_v, sem):
            base = chunk_base(c)
            cps = [
                pltpu.async_copy(src_hbm.at[pl.ds(base, CHUNK)], s_v, sem),
                pltpu.async_copy(dst_hbm.at[pl.ds(base, CHUNK)], d_v, sem),
                pltpu.async_copy(ew_hbm.at[pl.ds(base, CHUNK)], w_v, sem),
            ]
            return cps

        def wait_idx(cps):
            for cp in cps:
                cp.wait()

        HALF = CHUNK // 2

        def mul_half(rows_v, ew_v, h):
            def mul_group(g16, c2):
                gbase = h * HALF + g16 * 16
                w16 = ew_v[pl.ds(gbase, 16)]
                for r in range(16):
                    e = gbase + r
                    wv = w16[r]
                    for j in range(D_OUT // 16):
                        sl = pl.ds(j * 16, 16)
                        rows_v[e, sl] = rows_v[e, sl] * wv
                return c2

            lax.fori_loop(0, HALF // 16, mul_group, 0)

        def scatter_chunk(rows_v, ew_v, dst_v, dsc_v, sem):
            for h in range(2):
                mul_half(rows_v, ew_v, h)
                for j in range(HALF // 16):
                    dsc_v[h, pl.ds(j * 16, 16)] = (
                        dst_v[pl.ds(h * HALF + j * 16, 16)])
                pltpu.async_copy(rows_v.at[pl.ds(h * HALF, HALF)],
                                 acc.at[dsc_v.at[h]], sem, add=True)

        def wait_scatter(rows_v, dsc_v, sem):
            for h in range(2):
                pltpu.make_async_copy(rows_v.at[pl.ds(h * HALF, HALF)],
                                      acc.at[dsc_v.at[h]], sem).wait()

        wait_idx(load_idx(0, src_a, dst_a, ew_a, ia))
        cp_ga = [pltpu.async_copy(sup_hbm.at[src_a], rows_a, ga)]
        cps_ib = load_idx(1, src_b, dst_b, ew_b, ib)

        zeros16 = jnp.zeros((16,), jnp.float32)

        def zero_body(r, carry):
            for j in range(D_OUT // 16):
                rows_b[r, pl.ds(j * 16, 16)] = zeros16
            return carry

        lax.fori_loop(0, CHUNK, zero_body, 0)
        row0 = sid * ROWS_PER_TILE
        for kk in range(ROWS_PER_TILE // CHUNK + 1):
            nrows = CHUNK if kk < ROWS_PER_TILE // CHUNK else ROWS_PER_TILE % CHUNK
            pltpu.sync_copy(rows_b.at[pl.ds(0, nrows)],
                            acc.at[pl.ds(row0 + kk * CHUNK, nrows)])

        @pl.when(sid == NUM_SUBCORES - 1)
        def _zero_tail():
            pltpu.sync_copy(rows_b.at[pl.ds(0, TAIL_ROWS)],
                            acc.at[pl.ds(TAIL_ROW0, TAIL_ROWS)])

        plsc.subcore_barrier()

        izeros16 = jnp.zeros((16,), jnp.int32)
        for h in range(2):
            for j in range(HALF // 16):
                dsc_b[h, pl.ds(j * 16, 16)] = izeros16
            pltpu.async_copy(rows_b.at[pl.ds(h * HALF, HALF)],
                             acc.at[dsc_b.at[h]], sb, add=True)

        def pair_body(j, carry):
            c_a = 2 * j
            c_b = c_a + 1
            wait_idx(cps_ib)
            wait_scatter(rows_b, dsc_b, sb)
            cp_gb = pltpu.async_copy(sup_hbm.at[src_b], rows_b, gb)
            wait_idx(cp_ga)
            scatter_chunk(rows_a, ew_a, dst_a, dsc_a, sa)
            cps_ia = load_idx(jnp.minimum(c_a + 2, cmax), src_a, dst_a,
                              ew_a, ia)
            cp_gb.wait()
            scatter_chunk(rows_b, ew_b, dst_b, dsc_b, sb)
            cps_ib2 = load_idx(jnp.minimum(c_b + 2, cmax), src_b, dst_b,
                               ew_b, ib)
            wait_scatter(rows_a, dsc_a, sa)
            wait_idx(cps_ia)
            cp_ga2 = pltpu.async_copy(
                sup_hbm.at[src_a], rows_a, ga)
            return carry

        lax.fori_loop(0, npairs, pair_body, 0)
        pltpu.make_async_copy(sup_hbm.at[src_a], rows_a, ga).wait()
        wait_scatter(rows_b, dsc_b, sb)
        pltpu.make_async_copy(src_hbm.at[pl.ds(0, CHUNK)], src_b, ib).wait()
        pltpu.make_async_copy(dst_hbm.at[pl.ds(0, CHUNK)], dst_b, ib).wait()
        pltpu.make_async_copy(ew_hbm.at[pl.ds(0, CHUNK)], ew_b, ib).wait()
        plsc.subcore_barrier()

        for kk in range(ROWS_PER_TILE // CHUNK + 1):
            nrows = CHUNK if kk < ROWS_PER_TILE // CHUNK else ROWS_PER_TILE % CHUNK
            r0 = row0 + kk * CHUNK
            pltpu.sync_copy(acc.at[pl.ds(r0, nrows)],
                            rows_a.at[pl.ds(0, nrows)])
            pltpu.sync_copy(rows_a.at[pl.ds(0, nrows)],
                            out_hbm.at[cid].at[pl.ds(r0, nrows)])

        @pl.when(sid == NUM_SUBCORES - 1)
        def _write_tail():
            pltpu.sync_copy(acc.at[pl.ds(TAIL_ROW0, TAIL_ROWS)],
                            rows_a.at[pl.ds(0, TAIL_ROWS)])
            pltpu.sync_copy(
                rows_a.at[pl.ds(0, TAIL_ROWS)],
                out_hbm.at[cid].at[pl.ds(TAIL_ROW0, TAIL_ROWS)])

    return k(sup, src, dst, ew)


def _combine_body(p_ref, o_ref):
    o_ref[...] = p_ref[0] + p_ref[1]


def _combine(partials):
    return pl.pallas_call(
        _combine_body,
        grid=(10,),
        in_specs=[pl.BlockSpec((NUM_CORES, N_NODES // 10, D_OUT),
                               lambda i: (0, i, 0))],
        out_specs=pl.BlockSpec((N_NODES // 10, D_OUT), lambda i: (i, 0)),
        out_shape=jax.ShapeDtypeStruct((N_NODES, D_OUT), jnp.float32),
    )(partials)


def kernel(x, edge_index, edge_weight, W):
    src = edge_index[0].astype(jnp.int32)
    dst = edge_index[1].astype(jnp.int32)
    sup = _support(x, W)
    return _combine(_sc_spmm(sup, src, dst, edge_weight))
